# baseline (speedup 1.0000x reference)
import functools

import jax
import jax.numpy as jnp
from jax import lax
from jax.experimental import pallas as pl
from jax.experimental.pallas import tpu as pltpu
from jax.experimental.pallas import tpu_sc as plsc

_CH = 128
_PAD = 128


def _make_emb_kernel(B, L, H, V):
    info = plsc.get_sparse_core_info()
    NC, NS, LN = info.num_cores, info.num_subcores, info.num_lanes
    NW = NC * NS
    T = B * L
    assert T % (NW * _CH) == 0 and H % LN == 0
    chunks_per_w = T // (NW * _CH)
    assert chunks_per_w % 2 == 0 and chunks_per_w >= 4

    mesh = plsc.VectorSubcoreMesh(core_axis_name="c", subcore_axis_name="s")

    @functools.partial(
        pl.kernel,
        out_type=jax.ShapeDtypeStruct((T // _CH, _CH, H), jnp.float32),
        mesh=mesh,
        scratch_types=[
            pltpu.VMEM((chunks_per_w * _CH,), jnp.int32),
            pltpu.VMEM((L, _PAD), jnp.float32),
            pltpu.VMEM((_CH, _PAD), jnp.float32),
            pltpu.VMEM((_CH, _PAD), jnp.float32),
            pltpu.VMEM((_CH, H), jnp.float32),
            pltpu.VMEM((_CH, H), jnp.float32),
            pltpu.SemaphoreType.DMA,
            pltpu.SemaphoreType.DMA,
            pltpu.SemaphoreType.DMA,
            pltpu.SemaphoreType.DMA,
        ],
        compiler_params=pltpu.CompilerParams(use_tc_tiling_on_sc=False),
    )
    def emb_kernel(x_hbm, emb_hbm, pos_hbm, out_hbm, idx_v, pos_v, buf_a,
                   buf_b, out_va, out_vb, sem_a, sem_b, sem_sa, sem_sb):
        wid = lax.axis_index("s") * NC + lax.axis_index("c")
        tok0 = wid * chunks_per_w * _CH
        pltpu.sync_copy(pos_hbm, pos_v)
        pltpu.sync_copy(x_hbm.at[pl.ds(tok0, chunks_per_w * _CH)], idx_v)

        def start_gather(j, buf, sem):
            off = pl.multiple_of(j * _CH, _CH)
            pltpu.async_copy(emb_hbm.at[idx_v.at[pl.ds(off, _CH)]], buf, sem)

        def wait_gather(buf, sem):
            pltpu.make_async_copy(
                emb_hbm.at[idx_v.at[pl.ds(0, _CH)]], buf, sem).wait()

        def wait_store(ov, ssem):
            pltpu.make_async_copy(ov, out_hbm.at[0], ssem).wait()

        def add_and_store(j, buf, ov, ssem):
            base = pl.multiple_of(tok0 + j * _CH, _CH)

            @plsc.parallel_loop(0, _CH, step=1, unroll=8)
            def add_body(r):
                p = lax.rem(base + r, L)
                for c4 in range(H // LN):
                    sl = pl.ds(c4 * LN, LN)
                    ov[r, sl] = buf[r, sl] + pos_v[p, sl]

            pltpu.async_copy(ov, out_hbm.at[wid * chunks_per_w + j], ssem)

        start_gather(0, buf_a, sem_a)

        def pair_body(j2, carry):
            j0 = 2 * j2

            @pl.when(j0 + 1 < chunks_per_w)
            def _():
                start_gather(j0 + 1, buf_b, sem_b)

            wait_gather(buf_a, sem_a)

            @pl.when(j0 >= 2)
            def _():
                wait_store(out_va, sem_sa)

            add_and_store(j0, buf_a, out_va, sem_sa)

            @pl.when(j0 + 2 < chunks_per_w)
            def _():
                start_gather(j0 + 2, buf_a, sem_a)

            @pl.when(j0 + 1 < chunks_per_w)
            def _():
                wait_gather(buf_b, sem_b)

                @pl.when(j0 >= 2)
                def _():
                    wait_store(out_vb, sem_sb)

                add_and_store(j0 + 1, buf_b, out_vb, sem_sb)

            return carry

        lax.fori_loop(0, chunks_per_w // 2, pair_body, 0)
        wait_store(out_va, sem_sa)
        wait_store(out_vb, sem_sb)

    return emb_kernel


def kernel(x, emb_table, pos_table):
    B, L = x.shape
    V, H = emb_table.shape
    emb_pad = jnp.pad(emb_table, ((0, 0), (0, _PAD - H)))
    pos_pad = jnp.pad(pos_table, ((0, 0), (0, _PAD - H)))
    x_flat = jnp.reshape(x.astype(jnp.int32), (-1,))
    emb = _make_emb_kernel(B, L, H, V)
    out = emb(x_flat, emb_pad, pos_pad)
    return jnp.reshape(out, (B, L, H))


# speedup vs baseline: 1.8800x; 1.8800x over previous
import functools

import jax
import jax.numpy as jnp
from jax import lax
from jax.experimental import pallas as pl
from jax.experimental.pallas import tpu as pltpu
from jax.experimental.pallas import tpu_sc as plsc

_CH = 128
_PAD = 128


def _make_emb_kernel(B, L, H, V):
    info = plsc.get_sparse_core_info()
    NC, NS, LN = info.num_cores, info.num_subcores, info.num_lanes
    NW = NC * NS
    T = B * L
    assert T % (NW * _CH) == 0 and H % LN == 0
    chunks_per_w = T // (NW * _CH)
    assert chunks_per_w % 2 == 0 and chunks_per_w >= 4

    mesh = plsc.VectorSubcoreMesh(core_axis_name="c", subcore_axis_name="s")

    @functools.partial(
        pl.kernel,
        out_type=jax.ShapeDtypeStruct((T // _CH, _CH, H), jnp.float32),
        mesh=mesh,
        scratch_types=[
            pltpu.VMEM((chunks_per_w * _CH,), jnp.int32),
            pltpu.VMEM((L, _PAD), jnp.float32),
            pltpu.VMEM((_CH, _PAD), jnp.float32),
            pltpu.VMEM((_CH, _PAD), jnp.float32),
            pltpu.VMEM((_CH, H), jnp.float32),
            pltpu.VMEM((_CH, H), jnp.float32),
            pltpu.SemaphoreType.DMA,
            pltpu.SemaphoreType.DMA,
            pltpu.SemaphoreType.DMA,
            pltpu.SemaphoreType.DMA,
        ],
    )
    def emb_kernel(x_hbm, emb_hbm, pos_hbm, out_hbm, idx_v, pos_v, buf_a,
                   buf_b, out_va, out_vb, sem_a, sem_b, sem_sa, sem_sb):
        wid = lax.axis_index("s") * NC + lax.axis_index("c")
        tok0 = wid * chunks_per_w * _CH
        pltpu.sync_copy(pos_hbm, pos_v)
        pltpu.sync_copy(x_hbm.at[pl.ds(tok0, chunks_per_w * _CH)], idx_v)

        def start_gather(j, buf, sem):
            off = pl.multiple_of(j * _CH, _CH)
            pltpu.async_copy(emb_hbm.at[idx_v.at[pl.ds(off, _CH)]], buf, sem)

        def wait_gather(buf, sem):
            pltpu.make_async_copy(
                emb_hbm.at[idx_v.at[pl.ds(0, _CH)]], buf, sem).wait()

        def wait_store(ov, ssem):
            pltpu.make_async_copy(ov, out_hbm.at[0], ssem).wait()

        def add_and_store(j, buf, ov, ssem):
            base = pl.multiple_of(tok0 + j * _CH, _CH)

            @plsc.parallel_loop(0, _CH, step=1, unroll=8)
            def add_body(r):
                p = lax.rem(base + r, L)
                for c4 in range(H // LN):
                    sl = pl.ds(c4 * LN, LN)
                    ov[r, sl] = buf[r, sl] + pos_v[p, sl]

            pltpu.async_copy(ov, out_hbm.at[wid * chunks_per_w + j], ssem)

        start_gather(0, buf_a, sem_a)

        def pair_body(j2, carry):
            j0 = 2 * j2

            @pl.when(j0 + 1 < chunks_per_w)
            def _():
                start_gather(j0 + 1, buf_b, sem_b)

            wait_gather(buf_a, sem_a)

            @pl.when(j0 >= 2)
            def _():
                wait_store(out_va, sem_sa)

            add_and_store(j0, buf_a, out_va, sem_sa)

            @pl.when(j0 + 2 < chunks_per_w)
            def _():
                start_gather(j0 + 2, buf_a, sem_a)

            @pl.when(j0 + 1 < chunks_per_w)
            def _():
                wait_gather(buf_b, sem_b)

                @pl.when(j0 >= 2)
                def _():
                    wait_store(out_vb, sem_sb)

                add_and_store(j0 + 1, buf_b, out_vb, sem_sb)

            return carry

        lax.fori_loop(0, chunks_per_w // 2, pair_body, 0)
        wait_store(out_va, sem_sa)
        wait_store(out_vb, sem_sb)

    return emb_kernel


def kernel(x, emb_table, pos_table):
    B, L = x.shape
    V, H = emb_table.shape
    emb_pad = jnp.pad(emb_table, ((0, 0), (0, _PAD - H)))
    pos_pad = jnp.pad(pos_table, ((0, 0), (0, _PAD - H)))
    x_flat = jnp.reshape(x.astype(jnp.int32), (-1,))
    emb = _make_emb_kernel(B, L, H, V)
    out = emb(x_flat, emb_pad, pos_pad)
    return jnp.reshape(out, (B, L, H))
